# baseline (speedup 1.0000x reference)
import functools

import jax
import jax.numpy as jnp
from jax import lax
from jax.experimental import pallas as pl
from jax.experimental.pallas import tpu as pltpu
from jax.experimental.pallas import tpu_sc as plsc

N = 10000
K = 16
C_IN = 128
C_ENC = 64
N_OBS = 32
C_HID = 96
GROUPS = 8
KERNEL_REACH = 0.1

NP_PAD = 10240
KNN_Q = 400
GIB_B = 400
ROW_B = 2000
B_PAD = 163840


def _sc_gather(table, idx, chunk):
    B = idx.shape[0]
    D = table.shape[1]
    info = plsc.get_sparse_core_info()
    nw = info.num_cores * info.num_subcores
    b_per_w = B // nw
    n_chunks = b_per_w // chunk
    assert b_per_w % chunk == 0 and chunk % 8 == 0 and b_per_w % 8 == 0

    mesh = plsc.VectorSubcoreMesh(core_axis_name="c", subcore_axis_name="s")

    @functools.partial(
        pl.kernel, mesh=mesh,
        out_type=jax.ShapeDtypeStruct((B, D), jnp.float32),
        scratch_types=[
            pltpu.VMEM((chunk,), jnp.int32),
            pltpu.VMEM((chunk,), jnp.int32),
            pltpu.VMEM((2, chunk, D), jnp.float32),
            pltpu.SemaphoreType.DMA,
            pltpu.SemaphoreType.DMA,
            pltpu.SemaphoreType.DMA,
        ],
        name="sc_gather",
    )
    def gather_k(tab_hbm, idx_hbm, out_hbm, idx_v0, idx_v1, rows_v,
                 sem_g, s_sem0, s_sem1):
        wid = lax.axis_index("s") * info.num_cores + lax.axis_index("c")
        base = wid * b_per_w
        idx_vs = (idx_v0, idx_v1)
        s_sems = (s_sem0, s_sem1)
        stores = [None, None]
        for c in range(n_chunks):
            b = c % 2
            off = base + c * chunk
            pltpu.sync_copy(idx_hbm.at[pl.ds(off, chunk)], idx_vs[b])
            if stores[b] is not None:
                stores[b].wait()
            pltpu.async_copy(tab_hbm.at[idx_vs[b]], rows_v.at[b], sem_g).wait()
            stores[b] = pltpu.async_copy(
                rows_v.at[b], out_hbm.at[pl.ds(off, chunk)], s_sems[b])
        for s in stores:
            if s is not None:
                s.wait()

    return gather_k(table, idx)


def _knn_body(coordT_ref, q_ref, idx_ref):
    cx = coordT_ref[0:1, :]
    cy = coordT_ref[1:2, :]
    cz = coordT_ref[2:3, :]
    q = q_ref[...]
    dx = q[:, 0:1] - cx
    dy = q[:, 1:2] - cy
    dz = q[:, 2:3] - cz
    d = dx * dx + dy * dy + dz * dz
    lane = lax.broadcasted_iota(jnp.int32, (KNN_Q, NP_PAD), 1)
    cols = []
    for _ in range(K):
        amin = jnp.argmin(d, axis=1).astype(jnp.int32).reshape(KNN_Q, 1)
        cols.append(amin)
        d = jnp.where(lane == amin, jnp.float32(3.0e30), d)
    idx_ref[...] = jnp.concatenate(cols, axis=1)


def _knn(coordT, coord):
    return pl.pallas_call(
        _knn_body,
        grid=(N // KNN_Q,),
        in_specs=[
            pl.BlockSpec((8, NP_PAD), lambda i: (0, 0)),
            pl.BlockSpec((KNN_Q, 3), lambda i: (i, 0)),
        ],
        out_specs=pl.BlockSpec((KNN_Q, K), lambda i: (i, 0)),
        out_shape=jax.ShapeDtypeStruct((N, K), jnp.int32),
    )(coordT, coord)


def _gib_body(coord_ref, fcnbr_ref, feat_ref,
              dirs_ref, Wenc_ref, benc_ref, W1_ref, b1_ref, W2_ref, b2_ref,
              xpre_ref, stats_ref, rel_ref):
    i = pl.program_id(0)
    B = GIB_B
    c_self = coord_ref[...]
    fcn = fcnbr_ref[...]
    rx = fcn[:, :, 128:129] - c_self[:, 0:1][:, :, None]
    ry = fcn[:, :, 129:130] - c_self[:, 1:2][:, :, None]
    rz = fcn[:, :, 130:131] - c_self[:, 2:3][:, :, None]
    rel_ref[...] = jnp.concatenate(
        [rx, ry, rz, jnp.zeros((B, K, 5), jnp.float32)], axis=2)
    gauss = jnp.exp(-(rx * rx + ry * ry + rz * rz) / (KERNEL_REACH ** 2))
    dirs = dirs_ref[...]
    resp = (rx * dirs[:, 0].reshape(1, 1, N_OBS)
            + ry * dirs[:, 1].reshape(1, 1, N_OBS)
            + rz * dirs[:, 2].reshape(1, 1, N_OBS))
    gib_feat = jnp.sum(gauss * resp, axis=1)
    fmean = jnp.mean(fcn[:, :, 0:C_IN], axis=1)
    nbr = jnp.dot(fmean, Wenc_ref[...], preferred_element_type=jnp.float32,
                  precision=lax.Precision.HIGHEST)
    nbr = nbr + benc_ref[...]
    gout = jnp.concatenate([nbr, gib_feat], axis=1)
    h1 = jax.nn.gelu(
        jnp.dot(gout, W1_ref[...], preferred_element_type=jnp.float32,
                  precision=lax.Precision.HIGHEST)
        + b1_ref[...])
    h = jnp.dot(h1, W2_ref[...], preferred_element_type=jnp.float32,
                  precision=lax.Precision.HIGHEST) + b2_ref[...]
    xp = feat_ref[...] + h
    xpre_ref[...] = xp
    s1 = jnp.sum(xp, axis=0, keepdims=True)
    s2 = jnp.sum(xp * xp, axis=0, keepdims=True)
    upd = jnp.concatenate([s1, s2, jnp.zeros((6, C_IN), jnp.float32)], axis=0)

    @pl.when(i == 0)
    def _():
        stats_ref[...] = upd

    @pl.when(i > 0)
    def _():
        stats_ref[...] = stats_ref[...] + upd


def _gib(coord, fcnbr, feat, dirs, Wenc, benc, W1, b1, W2, b2):
    g = N // GIB_B
    return pl.pallas_call(
        _gib_body,
        grid=(g,),
        in_specs=[
            pl.BlockSpec((GIB_B, 3), lambda i: (i, 0)),
            pl.BlockSpec((GIB_B, K, 2 * C_IN), lambda i: (i, 0, 0)),
            pl.BlockSpec((GIB_B, C_IN), lambda i: (i, 0)),
            pl.BlockSpec((N_OBS, 3), lambda i: (0, 0)),
            pl.BlockSpec((C_IN, C_ENC), lambda i: (0, 0)),
            pl.BlockSpec((1, C_ENC), lambda i: (0, 0)),
            pl.BlockSpec((C_HID, C_HID), lambda i: (0, 0)),
            pl.BlockSpec((1, C_HID), lambda i: (0, 0)),
            pl.BlockSpec((C_HID, C_IN), lambda i: (0, 0)),
            pl.BlockSpec((1, C_IN), lambda i: (0, 0)),
        ],
        out_specs=[
            pl.BlockSpec((GIB_B, C_IN), lambda i: (i, 0)),
            pl.BlockSpec((8, C_IN), lambda i: (0, 0)),
            pl.BlockSpec((GIB_B, K, 8), lambda i: (i, 0, 0)),
        ],
        out_shape=[
            jax.ShapeDtypeStruct((N, C_IN), jnp.float32),
            jax.ShapeDtypeStruct((8, C_IN), jnp.float32),
            jax.ShapeDtypeStruct((N, K, 8), jnp.float32),
        ],
    )(coord, fcnbr, feat, dirs, Wenc, benc, W1, b1, W2, b2)


def _bn(v, stats_ref, g_ref, b_ref):
    mu = stats_ref[0:1, :] / N
    var = stats_ref[1:2, :] / N - mu * mu
    return (v - mu) * lax.rsqrt(var + 1e-5) * g_ref[...] + b_ref[...]


def _qkv_body(xpre_ref, stats_ref, g1_ref, be1_ref, Wqkv_ref, bqkv_ref,
              x_ref, q_ref, kv_ref):
    x = jax.nn.gelu(_bn(xpre_ref[...], stats_ref, g1_ref, be1_ref))
    x_ref[...] = x
    qkv = jnp.dot(x, Wqkv_ref[...], preferred_element_type=jnp.float32,
                  precision=lax.Precision.HIGHEST)
    qkv = qkv + bqkv_ref[...]
    q_ref[...] = qkv[:, 0:C_IN]
    kv_ref[...] = qkv[:, C_IN:3 * C_IN]


def _qkv(xpre, stats, g1, be1, Wqkv, bqkv):
    g = N // ROW_B
    return pl.pallas_call(
        _qkv_body,
        grid=(g,),
        in_specs=[
            pl.BlockSpec((ROW_B, C_IN), lambda i: (i, 0)),
            pl.BlockSpec((8, C_IN), lambda i: (0, 0)),
            pl.BlockSpec((1, C_IN), lambda i: (0, 0)),
            pl.BlockSpec((1, C_IN), lambda i: (0, 0)),
            pl.BlockSpec((C_IN, 3 * C_IN), lambda i: (0, 0)),
            pl.BlockSpec((1, 3 * C_IN), lambda i: (0, 0)),
        ],
        out_specs=[
            pl.BlockSpec((ROW_B, C_IN), lambda i: (i, 0)),
            pl.BlockSpec((ROW_B, C_IN), lambda i: (i, 0)),
            pl.BlockSpec((ROW_B, 2 * C_IN), lambda i: (i, 0)),
        ],
        out_shape=[
            jax.ShapeDtypeStruct((N, C_IN), jnp.float32),
            jax.ShapeDtypeStruct((N, C_IN), jnp.float32),
            jax.ShapeDtypeStruct((N, 2 * C_IN), jnp.float32),
        ],
    )(xpre, stats, g1, be1, Wqkv, bqkv)


def _attn_body(rel_ref, q_ref, kvnbr_ref,
               Wpe_ref, bpe_ref, Wwe_ref, bwe_ref, Wo_ref, bo_ref,
               Wfc_ref, bfc_ref, fcp_ref, stats_ref):
    i = pl.program_id(0)
    B = GIB_B
    kvn = kvnbr_ref[...]
    rel = rel_ref[...]
    rx = rel[:, :, 0:1]
    ry = rel[:, :, 1:2]
    rz = rel[:, :, 2:3]
    Wpe = Wpe_ref[...]
    pe = (rx * Wpe[0, :].reshape(1, 1, C_IN)
          + ry * Wpe[1, :].reshape(1, 1, C_IN)
          + rz * Wpe[2, :].reshape(1, 1, C_IN)
          + bpe_ref[...].reshape(1, 1, C_IN))
    r = q_ref[...][:, None, :] - kvn[:, :, 0:C_IN] + pe
    logits = jnp.dot(r.reshape(B * K, C_IN), Wwe_ref[...],
                     preferred_element_type=jnp.float32,
                  precision=lax.Precision.HIGHEST)
    logits = (logits + bwe_ref[...]).reshape(B, K, GROUPS)
    m = jnp.max(logits, axis=1, keepdims=True)
    e = jnp.exp(logits - m)
    attn = e / jnp.sum(e, axis=1, keepdims=True)
    lane_i = lax.broadcasted_iota(jnp.int32, (GROUPS, C_IN), 1)
    grp_i = lax.broadcasted_iota(jnp.int32, (GROUPS, C_IN), 0)
    expand = (lane_i // (C_IN // GROUPS) == grp_i).astype(jnp.float32)
    attn_rep = jnp.dot(attn.reshape(B * K, GROUPS), expand,
                       preferred_element_type=jnp.float32,
                  precision=lax.Precision.HIGHEST).reshape(B, K, C_IN)
    agg = jnp.sum(attn_rep * (kvn[:, :, C_IN:2 * C_IN] + pe), axis=1)
    agg = jnp.dot(agg, Wo_ref[...], preferred_element_type=jnp.float32,
                  precision=lax.Precision.HIGHEST)
    agg = agg + bo_ref[...]
    fcp = jnp.dot(agg, Wfc_ref[...], preferred_element_type=jnp.float32,
                  precision=lax.Precision.HIGHEST)
    fcp = fcp + bfc_ref[...]
    fcp_ref[...] = fcp
    s1 = jnp.sum(fcp, axis=0, keepdims=True)
    s2 = jnp.sum(fcp * fcp, axis=0, keepdims=True)
    upd = jnp.concatenate([s1, s2, jnp.zeros((6, C_IN), jnp.float32)], axis=0)

    @pl.when(i == 0)
    def _():
        stats_ref[...] = upd

    @pl.when(i > 0)
    def _():
        stats_ref[...] = stats_ref[...] + upd


def _attn(rel, q, kvnbr, Wpe, bpe, Wwe, bwe, Wo, bo, Wfc, bfc):
    g = N // GIB_B
    return pl.pallas_call(
        _attn_body,
        grid=(g,),
        in_specs=[
            pl.BlockSpec((GIB_B, K, 8), lambda i: (i, 0, 0)),
            pl.BlockSpec((GIB_B, C_IN), lambda i: (i, 0)),
            pl.BlockSpec((GIB_B, K, 2 * C_IN), lambda i: (i, 0, 0)),
            pl.BlockSpec((3, C_IN), lambda i: (0, 0)),
            pl.BlockSpec((1, C_IN), lambda i: (0, 0)),
            pl.BlockSpec((C_IN, GROUPS), lambda i: (0, 0)),
            pl.BlockSpec((1, GROUPS), lambda i: (0, 0)),
            pl.BlockSpec((C_IN, C_IN), lambda i: (0, 0)),
            pl.BlockSpec((1, C_IN), lambda i: (0, 0)),
            pl.BlockSpec((C_IN, C_IN), lambda i: (0, 0)),
            pl.BlockSpec((1, C_IN), lambda i: (0, 0)),
        ],
        out_specs=[
            pl.BlockSpec((GIB_B, C_IN), lambda i: (i, 0)),
            pl.BlockSpec((8, C_IN), lambda i: (0, 0)),
        ],
        out_shape=[
            jax.ShapeDtypeStruct((N, C_IN), jnp.float32),
            jax.ShapeDtypeStruct((8, C_IN), jnp.float32),
        ],
    )(rel, q, kvnbr, Wpe, bpe, Wwe, bwe, Wo, bo, Wfc, bfc)


def _sota_body(x_ref, fcp_ref, stats_ref, gfc_ref, befc_ref,
               Ws1_ref, bs1_ref, Ws2_ref, bs2_ref, zp_ref, stats3_ref):
    i = pl.program_id(0)
    y = x_ref[...] + jax.nn.gelu(_bn(fcp_ref[...], stats_ref, gfc_ref, befc_ref))
    z1 = jax.nn.gelu(
        jnp.dot(y, Ws1_ref[...], preferred_element_type=jnp.float32,
                  precision=lax.Precision.HIGHEST)
        + bs1_ref[...])
    zp = jnp.dot(z1, Ws2_ref[...], preferred_element_type=jnp.float32,
                  precision=lax.Precision.HIGHEST)
    zp = zp + bs2_ref[...]
    zp_ref[...] = zp
    s1 = jnp.sum(zp, axis=0, keepdims=True)
    s2 = jnp.sum(zp * zp, axis=0, keepdims=True)
    upd = jnp.concatenate([s1, s2, jnp.zeros((6, C_IN), jnp.float32)], axis=0)

    @pl.when(i == 0)
    def _():
        stats3_ref[...] = upd

    @pl.when(i > 0)
    def _():
        stats3_ref[...] = stats3_ref[...] + upd


def _sota(x, fcp, stats, gfc, befc, Ws1, bs1, Ws2, bs2):
    g = N // ROW_B
    return pl.pallas_call(
        _sota_body,
        grid=(g,),
        in_specs=[
            pl.BlockSpec((ROW_B, C_IN), lambda i: (i, 0)),
            pl.BlockSpec((ROW_B, C_IN), lambda i: (i, 0)),
            pl.BlockSpec((8, C_IN), lambda i: (0, 0)),
            pl.BlockSpec((1, C_IN), lambda i: (0, 0)),
            pl.BlockSpec((1, C_IN), lambda i: (0, 0)),
            pl.BlockSpec((C_IN, C_IN), lambda i: (0, 0)),
            pl.BlockSpec((1, C_IN), lambda i: (0, 0)),
            pl.BlockSpec((C_IN, C_IN), lambda i: (0, 0)),
            pl.BlockSpec((1, C_IN), lambda i: (0, 0)),
        ],
        out_specs=[
            pl.BlockSpec((ROW_B, C_IN), lambda i: (i, 0)),
            pl.BlockSpec((8, C_IN), lambda i: (0, 0)),
        ],
        out_shape=[
            jax.ShapeDtypeStruct((N, C_IN), jnp.float32),
            jax.ShapeDtypeStruct((8, C_IN), jnp.float32),
        ],
    )(x, fcp, stats, gfc, befc, Ws1, bs1, Ws2, bs2)


def _final_body(zp_ref, stats_ref, g2_ref, be2_ref, z_ref):
    z_ref[...] = jax.nn.gelu(_bn(zp_ref[...], stats_ref, g2_ref, be2_ref))


def _final(zp, stats, g2, be2):
    g = N // ROW_B
    return pl.pallas_call(
        _final_body,
        grid=(g,),
        in_specs=[
            pl.BlockSpec((ROW_B, C_IN), lambda i: (i, 0)),
            pl.BlockSpec((8, C_IN), lambda i: (0, 0)),
            pl.BlockSpec((1, C_IN), lambda i: (0, 0)),
            pl.BlockSpec((1, C_IN), lambda i: (0, 0)),
        ],
        out_specs=pl.BlockSpec((ROW_B, C_IN), lambda i: (i, 0)),
        out_shape=jax.ShapeDtypeStruct((N, C_IN), jnp.float32),
    )(zp, stats, g2, be2)


def kernel(coord, feat, offset, gib_dirs, W_enc, b_enc, W1, b1, W2, b2, g1, be1,
           Wqkv, bqkv, Wpe, bpe, Wwe, bwe, Wo, bo, Wfc, bfc, gfc, befc,
           Ws1, bs1, Ws2, bs2, g2, be2):
    r1 = lambda v: v.reshape(1, -1)

    coordT = jnp.full((8, NP_PAD), 1.0e8, jnp.float32)
    coordT = coordT.at[0:3, 0:N].set(coord.T)
    idx = _knn(coordT, coord)
    idx_flat = jnp.zeros((B_PAD,), jnp.int32).at[0:N * K].set(idx.reshape(-1))
    n_rows = B_PAD // K

    fc = jnp.concatenate([feat, jnp.pad(coord, ((0, 0), (0, C_IN - 3)))], axis=1)
    fcnbr = _sc_gather(fc, idx_flat, chunk=160).reshape(n_rows, K, 2 * C_IN)

    xpre, stats1, rel = _gib(coord, fcnbr, feat, gib_dirs,
                             W_enc, r1(b_enc), W1, r1(b1), W2, r1(b2))
    x, q, kv = _qkv(xpre, stats1, r1(g1), r1(be1), Wqkv, r1(bqkv))

    kvnbr = _sc_gather(kv, idx_flat, chunk=160).reshape(n_rows, K, 2 * C_IN)

    fcp, stats2 = _attn(rel, q, kvnbr, Wpe, r1(bpe),
                        Wwe, r1(bwe), Wo, r1(bo), Wfc, r1(bfc))
    zp, stats3 = _sota(x, fcp, stats2, r1(gfc), r1(befc),
                       Ws1, r1(bs1), Ws2, r1(bs2))
    z = _final(zp, stats3, r1(g2), r1(be2))
    return (coord, z, offset)


# speedup vs baseline: 1.0995x; 1.0995x over previous
import functools

import jax
import jax.numpy as jnp
from jax import lax
from jax.experimental import pallas as pl
from jax.experimental.pallas import tpu as pltpu
from jax.experimental.pallas import tpu_sc as plsc

N = 10000
K = 16
C_IN = 128
C_ENC = 64
N_OBS = 32
C_HID = 96
GROUPS = 8
KERNEL_REACH = 0.1

NP_PAD = 10240
KNN_Q = 200
GIB_B = 400
ROW_B = 2000
B_PAD = 163840


def _sc_gather(table, idx, chunk):
    B = idx.shape[0]
    D = table.shape[1]
    info = plsc.get_sparse_core_info()
    nw = info.num_cores * info.num_subcores
    b_per_w = B // nw
    n_chunks = b_per_w // chunk
    assert b_per_w % chunk == 0 and chunk % 8 == 0 and b_per_w % 8 == 0

    mesh = plsc.VectorSubcoreMesh(core_axis_name="c", subcore_axis_name="s")

    @functools.partial(
        pl.kernel, mesh=mesh,
        out_type=jax.ShapeDtypeStruct((B, D), jnp.float32),
        scratch_types=[
            pltpu.VMEM((chunk,), jnp.int32),
            pltpu.VMEM((chunk,), jnp.int32),
            pltpu.VMEM((2, chunk, D), jnp.float32),
            pltpu.SemaphoreType.DMA,
            pltpu.SemaphoreType.DMA,
            pltpu.SemaphoreType.DMA,
        ],
        name="sc_gather",
    )
    def gather_k(tab_hbm, idx_hbm, out_hbm, idx_v0, idx_v1, rows_v,
                 sem_g, s_sem0, s_sem1):
        wid = lax.axis_index("s") * info.num_cores + lax.axis_index("c")
        base = wid * b_per_w
        idx_vs = (idx_v0, idx_v1)
        s_sems = (s_sem0, s_sem1)
        stores = [None, None]
        for c in range(n_chunks):
            b = c % 2
            off = base + c * chunk
            pltpu.sync_copy(idx_hbm.at[pl.ds(off, chunk)], idx_vs[b])
            if stores[b] is not None:
                stores[b].wait()
            pltpu.async_copy(tab_hbm.at[idx_vs[b]], rows_v.at[b], sem_g).wait()
            stores[b] = pltpu.async_copy(
                rows_v.at[b], out_hbm.at[pl.ds(off, chunk)], s_sems[b])
        for s in stores:
            if s is not None:
                s.wait()

    return gather_k(table, idx)


def _knn_body(coordT_ref, q_ref, idx_ref):
    cx = coordT_ref[0:1, :]
    cy = coordT_ref[1:2, :]
    cz = coordT_ref[2:3, :]
    q = q_ref[...]
    dx = q[:, 0:1] - cx
    dy = q[:, 1:2] - cy
    dz = q[:, 2:3] - cz
    d = dx * dx + dy * dy + dz * dz
    lane = lax.broadcasted_iota(jnp.int32, (KNN_Q, NP_PAD), 1)
    cols = []
    for _ in range(K):
        amin = jnp.argmin(d, axis=1).astype(jnp.int32).reshape(KNN_Q, 1)
        cols.append(amin)
        d = jnp.where(lane == amin, jnp.float32(3.0e30), d)
    idx_ref[...] = jnp.concatenate(cols, axis=1)


def _knn(coordT, coord):
    return pl.pallas_call(
        _knn_body,
        grid=(N // KNN_Q,),
        in_specs=[
            pl.BlockSpec((8, NP_PAD), lambda i: (0, 0)),
            pl.BlockSpec((KNN_Q, 3), lambda i: (i, 0)),
        ],
        out_specs=pl.BlockSpec((KNN_Q, K), lambda i: (i, 0)),
        out_shape=jax.ShapeDtypeStruct((N, K), jnp.int32),
    )(coordT, coord)


def _gib_body(coord_ref, fcnbr_ref, feat_ref,
              dirs_ref, Wenc_ref, benc_ref, W1_ref, b1_ref, W2_ref, b2_ref,
              xpre_ref, stats_ref, rel_ref):
    i = pl.program_id(0)
    B = GIB_B
    c_self = coord_ref[...]
    fcn = fcnbr_ref[...]
    rx = fcn[:, :, 128:129] - c_self[:, 0:1][:, :, None]
    ry = fcn[:, :, 129:130] - c_self[:, 1:2][:, :, None]
    rz = fcn[:, :, 130:131] - c_self[:, 2:3][:, :, None]
    rel_ref[...] = jnp.concatenate(
        [rx, ry, rz, jnp.ones((B, K, 1), jnp.float32),
         jnp.zeros((B, K, 4), jnp.float32)], axis=2)
    gauss = jnp.exp(-(rx * rx + ry * ry + rz * rz) / (KERNEL_REACH ** 2))
    dirs = dirs_ref[...]
    resp = (rx * dirs[:, 0].reshape(1, 1, N_OBS)
            + ry * dirs[:, 1].reshape(1, 1, N_OBS)
            + rz * dirs[:, 2].reshape(1, 1, N_OBS))
    gib_feat = jnp.sum(gauss * resp, axis=1)
    fmean = jnp.mean(fcn[:, :, 0:C_IN], axis=1)
    nbr = jnp.dot(fmean, Wenc_ref[...], preferred_element_type=jnp.float32,
                  precision=lax.Precision.HIGHEST)
    nbr = nbr + benc_ref[...]
    gout = jnp.concatenate([nbr, gib_feat], axis=1)
    h1 = jax.nn.gelu(
        jnp.dot(gout, W1_ref[...], preferred_element_type=jnp.float32,
                  precision=lax.Precision.HIGHEST)
        + b1_ref[...])
    h = jnp.dot(h1, W2_ref[...], preferred_element_type=jnp.float32,
                  precision=lax.Precision.HIGHEST) + b2_ref[...]
    xp = feat_ref[...] + h
    xpre_ref[...] = xp
    s1 = jnp.sum(xp, axis=0, keepdims=True)
    s2 = jnp.sum(xp * xp, axis=0, keepdims=True)
    upd = jnp.concatenate([s1, s2, jnp.zeros((6, C_IN), jnp.float32)], axis=0)

    @pl.when(i == 0)
    def _():
        stats_ref[...] = upd

    @pl.when(i > 0)
    def _():
        stats_ref[...] = stats_ref[...] + upd


def _gib(coord, fcnbr, feat, dirs, Wenc, benc, W1, b1, W2, b2):
    g = N // GIB_B
    return pl.pallas_call(
        _gib_body,
        grid=(g,),
        in_specs=[
            pl.BlockSpec((GIB_B, 3), lambda i: (i, 0)),
            pl.BlockSpec((GIB_B, K, 2 * C_IN), lambda i: (i, 0, 0)),
            pl.BlockSpec((GIB_B, C_IN), lambda i: (i, 0)),
            pl.BlockSpec((N_OBS, 3), lambda i: (0, 0)),
            pl.BlockSpec((C_IN, C_ENC), lambda i: (0, 0)),
            pl.BlockSpec((1, C_ENC), lambda i: (0, 0)),
            pl.BlockSpec((C_HID, C_HID), lambda i: (0, 0)),
            pl.BlockSpec((1, C_HID), lambda i: (0, 0)),
            pl.BlockSpec((C_HID, C_IN), lambda i: (0, 0)),
            pl.BlockSpec((1, C_IN), lambda i: (0, 0)),
        ],
        out_specs=[
            pl.BlockSpec((GIB_B, C_IN), lambda i: (i, 0)),
            pl.BlockSpec((8, C_IN), lambda i: (0, 0)),
            pl.BlockSpec((GIB_B, K, 8), lambda i: (i, 0, 0)),
        ],
        out_shape=[
            jax.ShapeDtypeStruct((N, C_IN), jnp.float32),
            jax.ShapeDtypeStruct((8, C_IN), jnp.float32),
            jax.ShapeDtypeStruct((N, K, 8), jnp.float32),
        ],
    )(coord, fcnbr, feat, dirs, Wenc, benc, W1, b1, W2, b2)


def _bn(v, stats_ref, g_ref, b_ref):
    mu = stats_ref[0:1, :] / N
    var = stats_ref[1:2, :] / N - mu * mu
    return (v - mu) * lax.rsqrt(var + 1e-5) * g_ref[...] + b_ref[...]


def _qkv_body(xpre_ref, stats_ref, g1_ref, be1_ref, Wqkv_ref, bqkv_ref,
              x_ref, q_ref, kv_ref):
    x = jax.nn.gelu(_bn(xpre_ref[...], stats_ref, g1_ref, be1_ref))
    x_ref[...] = x
    qkv = jnp.dot(x, Wqkv_ref[...], preferred_element_type=jnp.float32,
                  precision=lax.Precision.HIGHEST)
    qkv = qkv + bqkv_ref[...]
    q_ref[...] = qkv[:, 0:C_IN]
    kv_ref[...] = qkv[:, C_IN:3 * C_IN]


def _qkv(xpre, stats, g1, be1, Wqkv, bqkv):
    g = N // ROW_B
    return pl.pallas_call(
        _qkv_body,
        grid=(g,),
        in_specs=[
            pl.BlockSpec((ROW_B, C_IN), lambda i: (i, 0)),
            pl.BlockSpec((8, C_IN), lambda i: (0, 0)),
            pl.BlockSpec((1, C_IN), lambda i: (0, 0)),
            pl.BlockSpec((1, C_IN), lambda i: (0, 0)),
            pl.BlockSpec((C_IN, 3 * C_IN), lambda i: (0, 0)),
            pl.BlockSpec((1, 3 * C_IN), lambda i: (0, 0)),
        ],
        out_specs=[
            pl.BlockSpec((ROW_B, C_IN), lambda i: (i, 0)),
            pl.BlockSpec((ROW_B, C_IN), lambda i: (i, 0)),
            pl.BlockSpec((ROW_B, 2 * C_IN), lambda i: (i, 0)),
        ],
        out_shape=[
            jax.ShapeDtypeStruct((N, C_IN), jnp.float32),
            jax.ShapeDtypeStruct((N, C_IN), jnp.float32),
            jax.ShapeDtypeStruct((N, 2 * C_IN), jnp.float32),
        ],
    )(xpre, stats, g1, be1, Wqkv, bqkv)


def _attn_body(rel_ref, q_ref, kvnbr_ref,
               Wpe8_ref, Wwe_ref, bwe_ref, Wo_ref, bo_ref,
               Wfc_ref, bfc_ref, fcp_ref, stats_ref):
    i = pl.program_id(0)
    B = GIB_B
    BK = GIB_B * K
    kvn = kvnbr_ref[...]
    pe = jnp.dot(rel_ref[...], Wpe8_ref[...],
                 preferred_element_type=jnp.float32,
                 precision=lax.Precision.HIGHEST)
    qw = jnp.dot(q_ref[...], Wwe_ref[...],
                 preferred_element_type=jnp.float32,
                 precision=lax.Precision.HIGHEST) + bwe_ref[...]
    lk = jnp.dot(pe - kvn[:, 0:C_IN], Wwe_ref[...],
                 preferred_element_type=jnp.float32,
                 precision=lax.Precision.HIGHEST)
    logits = lk.reshape(B, K, GROUPS) + qw[:, None, :]
    m = jnp.max(logits, axis=1, keepdims=True)
    e = jnp.exp(logits - m)
    attn = e / jnp.sum(e, axis=1, keepdims=True)
    lane_i = lax.broadcasted_iota(jnp.int32, (GROUPS, C_IN), 1)
    grp_i = lax.broadcasted_iota(jnp.int32, (GROUPS, C_IN), 0)
    expand = (lane_i // (C_IN // GROUPS) == grp_i).astype(jnp.float32)
    attn_rep = jnp.dot(attn.reshape(BK, GROUPS), expand,
                       preferred_element_type=jnp.float32,
                       precision=lax.Precision.HIGHEST)
    prod = attn_rep * (kvn[:, C_IN:2 * C_IN] + pe)
    agg = jnp.sum(prod.reshape(B, K, C_IN), axis=1)
    agg = jnp.dot(agg, Wo_ref[...], preferred_element_type=jnp.float32,
                  precision=lax.Precision.HIGHEST)
    agg = agg + bo_ref[...]
    fcp = jnp.dot(agg, Wfc_ref[...], preferred_element_type=jnp.float32,
                  precision=lax.Precision.HIGHEST)
    fcp = fcp + bfc_ref[...]
    fcp_ref[...] = fcp
    s1 = jnp.sum(fcp, axis=0, keepdims=True)
    s2 = jnp.sum(fcp * fcp, axis=0, keepdims=True)
    upd = jnp.concatenate([s1, s2, jnp.zeros((6, C_IN), jnp.float32)], axis=0)

    @pl.when(i == 0)
    def _():
        stats_ref[...] = upd

    @pl.when(i > 0)
    def _():
        stats_ref[...] = stats_ref[...] + upd


def _attn(rel_flat, q, kvnbr_flat, Wpe8, Wwe, bwe, Wo, bo, Wfc, bfc):
    g = N // GIB_B
    BK = GIB_B * K
    return pl.pallas_call(
        _attn_body,
        grid=(g,),
        in_specs=[
            pl.BlockSpec((BK, 8), lambda i: (i, 0)),
            pl.BlockSpec((GIB_B, C_IN), lambda i: (i, 0)),
            pl.BlockSpec((BK, 2 * C_IN), lambda i: (i, 0)),
            pl.BlockSpec((8, C_IN), lambda i: (0, 0)),
            pl.BlockSpec((C_IN, GROUPS), lambda i: (0, 0)),
            pl.BlockSpec((1, GROUPS), lambda i: (0, 0)),
            pl.BlockSpec((C_IN, C_IN), lambda i: (0, 0)),
            pl.BlockSpec((1, C_IN), lambda i: (0, 0)),
            pl.BlockSpec((C_IN, C_IN), lambda i: (0, 0)),
            pl.BlockSpec((1, C_IN), lambda i: (0, 0)),
        ],
        out_specs=[
            pl.BlockSpec((GIB_B, C_IN), lambda i: (i, 0)),
            pl.BlockSpec((8, C_IN), lambda i: (0, 0)),
        ],
        out_shape=[
            jax.ShapeDtypeStruct((N, C_IN), jnp.float32),
            jax.ShapeDtypeStruct((8, C_IN), jnp.float32),
        ],
    )(rel_flat, q, kvnbr_flat, Wpe8, Wwe, bwe, Wo, bo, Wfc, bfc)


def _sota_body(x_ref, fcp_ref, stats_ref, gfc_ref, befc_ref,
               Ws1_ref, bs1_ref, Ws2_ref, bs2_ref, zp_ref, stats3_ref):
    i = pl.program_id(0)
    y = x_ref[...] + jax.nn.gelu(_bn(fcp_ref[...], stats_ref, gfc_ref, befc_ref))
    z1 = jax.nn.gelu(
        jnp.dot(y, Ws1_ref[...], preferred_element_type=jnp.float32,
                  precision=lax.Precision.HIGHEST)
        + bs1_ref[...])
    zp = jnp.dot(z1, Ws2_ref[...], preferred_element_type=jnp.float32,
                  precision=lax.Precision.HIGHEST)
    zp = zp + bs2_ref[...]
    zp_ref[...] = zp
    s1 = jnp.sum(zp, axis=0, keepdims=True)
    s2 = jnp.sum(zp * zp, axis=0, keepdims=True)
    upd = jnp.concatenate([s1, s2, jnp.zeros((6, C_IN), jnp.float32)], axis=0)

    @pl.when(i == 0)
    def _():
        stats3_ref[...] = upd

    @pl.when(i > 0)
    def _():
        stats3_ref[...] = stats3_ref[...] + upd


def _sota(x, fcp, stats, gfc, befc, Ws1, bs1, Ws2, bs2):
    g = N // ROW_B
    return pl.pallas_call(
        _sota_body,
        grid=(g,),
        in_specs=[
            pl.BlockSpec((ROW_B, C_IN), lambda i: (i, 0)),
            pl.BlockSpec((ROW_B, C_IN), lambda i: (i, 0)),
            pl.BlockSpec((8, C_IN), lambda i: (0, 0)),
            pl.BlockSpec((1, C_IN), lambda i: (0, 0)),
            pl.BlockSpec((1, C_IN), lambda i: (0, 0)),
            pl.BlockSpec((C_IN, C_IN), lambda i: (0, 0)),
            pl.BlockSpec((1, C_IN), lambda i: (0, 0)),
            pl.BlockSpec((C_IN, C_IN), lambda i: (0, 0)),
            pl.BlockSpec((1, C_IN), lambda i: (0, 0)),
        ],
        out_specs=[
            pl.BlockSpec((ROW_B, C_IN), lambda i: (i, 0)),
            pl.BlockSpec((8, C_IN), lambda i: (0, 0)),
        ],
        out_shape=[
            jax.ShapeDtypeStruct((N, C_IN), jnp.float32),
            jax.ShapeDtypeStruct((8, C_IN), jnp.float32),
        ],
    )(x, fcp, stats, gfc, befc, Ws1, bs1, Ws2, bs2)


def _final_body(zp_ref, stats_ref, g2_ref, be2_ref, z_ref):
    z_ref[...] = jax.nn.gelu(_bn(zp_ref[...], stats_ref, g2_ref, be2_ref))


def _final(zp, stats, g2, be2):
    g = N // ROW_B
    return pl.pallas_call(
        _final_body,
        grid=(g,),
        in_specs=[
            pl.BlockSpec((ROW_B, C_IN), lambda i: (i, 0)),
            pl.BlockSpec((8, C_IN), lambda i: (0, 0)),
            pl.BlockSpec((1, C_IN), lambda i: (0, 0)),
            pl.BlockSpec((1, C_IN), lambda i: (0, 0)),
        ],
        out_specs=pl.BlockSpec((ROW_B, C_IN), lambda i: (i, 0)),
        out_shape=jax.ShapeDtypeStruct((N, C_IN), jnp.float32),
    )(zp, stats, g2, be2)


def kernel(coord, feat, offset, gib_dirs, W_enc, b_enc, W1, b1, W2, b2, g1, be1,
           Wqkv, bqkv, Wpe, bpe, Wwe, bwe, Wo, bo, Wfc, bfc, gfc, befc,
           Ws1, bs1, Ws2, bs2, g2, be2):
    r1 = lambda v: v.reshape(1, -1)

    coordT = jnp.full((8, NP_PAD), 1.0e8, jnp.float32)
    coordT = coordT.at[0:3, 0:N].set(coord.T)
    idx = _knn(coordT, coord)
    idx_flat = jnp.zeros((B_PAD,), jnp.int32).at[0:N * K].set(idx.reshape(-1))
    n_rows = B_PAD // K

    fc = jnp.concatenate([feat, jnp.pad(coord, ((0, 0), (0, C_IN - 3)))], axis=1)
    fcnbr = _sc_gather(fc, idx_flat, chunk=160).reshape(n_rows, K, 2 * C_IN)

    xpre, stats1, rel = _gib(coord, fcnbr, feat, gib_dirs,
                             W_enc, r1(b_enc), W1, r1(b1), W2, r1(b2))
    x, q, kv = _qkv(xpre, stats1, r1(g1), r1(be1), Wqkv, r1(bqkv))

    kvnbr_flat = _sc_gather(kv, idx_flat, chunk=160)

    Wpe8 = jnp.concatenate(
        [Wpe, bpe.reshape(1, C_IN), jnp.zeros((4, C_IN), jnp.float32)], axis=0)
    fcp, stats2 = _attn(rel.reshape(N * K, 8), q, kvnbr_flat, Wpe8,
                        Wwe, r1(bwe), Wo, r1(bo), Wfc, r1(bfc))
    zp, stats3 = _sota(x, fcp, stats2, r1(gfc), r1(befc),
                       Ws1, r1(bs1), Ws2, r1(bs2))
    z = _final(zp, stats3, r1(g2), r1(be2))
    return (coord, z, offset)


# speedup vs baseline: 1.2144x; 1.1045x over previous
import functools

import jax
import jax.numpy as jnp
from jax import lax
from jax.experimental import pallas as pl
from jax.experimental.pallas import tpu as pltpu
from jax.experimental.pallas import tpu_sc as plsc

N = 10000
K = 16
C_IN = 128
C_ENC = 64
N_OBS = 32
C_HID = 96
GROUPS = 8
KERNEL_REACH = 0.1

NP_PAD = 10240
KNN_Q = 200
GIB_B = 200
ROW_B = 1000
NH = N // 2
HB_PAD = 81920


def _sc_gather(table, idx, chunk):
    B = idx.shape[0]
    D = table.shape[1]
    info = plsc.get_sparse_core_info()
    nw = info.num_cores * info.num_subcores
    b_per_w = B // nw
    n_chunks = b_per_w // chunk
    assert b_per_w % chunk == 0 and chunk % 8 == 0 and b_per_w % 8 == 0

    mesh = plsc.VectorSubcoreMesh(core_axis_name="c", subcore_axis_name="s")

    @functools.partial(
        pl.kernel, mesh=mesh,
        out_type=jax.ShapeDtypeStruct((B, D), jnp.float32),
        scratch_types=[
            pltpu.VMEM((chunk,), jnp.int32),
            pltpu.VMEM((chunk,), jnp.int32),
            pltpu.VMEM((2, chunk, D), jnp.float32),
            pltpu.SemaphoreType.DMA,
            pltpu.SemaphoreType.DMA,
            pltpu.SemaphoreType.DMA,
        ],
        name="sc_gather",
    )
    def gather_k(tab_hbm, idx_hbm, out_hbm, idx_v0, idx_v1, rows_v,
                 sem_g, s_sem0, s_sem1):
        wid = lax.axis_index("s") * info.num_cores + lax.axis_index("c")
        base = wid * b_per_w
        idx_vs = (idx_v0, idx_v1)
        s_sems = (s_sem0, s_sem1)
        stores = [None, None]
        for c in range(n_chunks):
            b = c % 2
            off = base + c * chunk
            pltpu.sync_copy(idx_hbm.at[pl.ds(off, chunk)], idx_vs[b])
            if stores[b] is not None:
                stores[b].wait()
            pltpu.async_copy(tab_hbm.at[idx_vs[b]], rows_v.at[b], sem_g).wait()
            stores[b] = pltpu.async_copy(
                rows_v.at[b], out_hbm.at[pl.ds(off, chunk)], s_sems[b])
        for s in stores:
            if s is not None:
                s.wait()

    return gather_k(table, idx)


def _knn_body(coordT_ref, q_ref, idx_ref):
    cx = coordT_ref[0:1, :]
    cy = coordT_ref[1:2, :]
    cz = coordT_ref[2:3, :]
    q = q_ref[...]
    dx = q[:, 0:1] - cx
    dy = q[:, 1:2] - cy
    dz = q[:, 2:3] - cz
    d = dx * dx + dy * dy + dz * dz
    lane = lax.broadcasted_iota(jnp.int32, (KNN_Q, NP_PAD), 1)
    cols = []
    for _ in range(K):
        amin = jnp.argmin(d, axis=1).astype(jnp.int32).reshape(KNN_Q, 1)
        cols.append(amin)
        d = jnp.where(lane == amin, jnp.float32(3.0e30), d)
    idx_ref[...] = jnp.concatenate(cols, axis=1)


def _knn(coordT, coord):
    M = coord.shape[0]
    return pl.pallas_call(
        _knn_body,
        grid=(M // KNN_Q,),
        in_specs=[
            pl.BlockSpec((8, NP_PAD), lambda i: (0, 0)),
            pl.BlockSpec((KNN_Q, 3), lambda i: (i, 0)),
        ],
        out_specs=pl.BlockSpec((KNN_Q, K), lambda i: (i, 0)),
        out_shape=jax.ShapeDtypeStruct((M, K), jnp.int32),
    )(coordT, coord)


def _gib_body(coord_ref, fcnbr_ref, feat_ref,
              dirs_ref, Wenc_ref, benc_ref, W1_ref, b1_ref, W2_ref, b2_ref,
              xpre_ref, stats_ref, rel_ref):
    i = pl.program_id(0)
    B = GIB_B
    c_self = coord_ref[...]
    fcn = fcnbr_ref[...]
    rx = fcn[:, :, 128:129] - c_self[:, 0:1][:, :, None]
    ry = fcn[:, :, 129:130] - c_self[:, 1:2][:, :, None]
    rz = fcn[:, :, 130:131] - c_self[:, 2:3][:, :, None]
    rel_ref[...] = jnp.concatenate(
        [rx, ry, rz, jnp.ones((B, K, 1), jnp.float32),
         jnp.zeros((B, K, 4), jnp.float32)], axis=2)
    gauss = jnp.exp(-(rx * rx + ry * ry + rz * rz) / (KERNEL_REACH ** 2))
    dirs = dirs_ref[...]
    resp = (rx * dirs[:, 0].reshape(1, 1, N_OBS)
            + ry * dirs[:, 1].reshape(1, 1, N_OBS)
            + rz * dirs[:, 2].reshape(1, 1, N_OBS))
    gib_feat = jnp.sum(gauss * resp, axis=1)
    fmean = jnp.mean(fcn[:, :, 0:C_IN], axis=1)
    nbr = jnp.dot(fmean, Wenc_ref[...], preferred_element_type=jnp.float32,
                  precision=lax.Precision.HIGHEST)
    nbr = nbr + benc_ref[...]
    gout = jnp.concatenate([nbr, gib_feat], axis=1)
    h1 = jax.nn.gelu(
        jnp.dot(gout, W1_ref[...], preferred_element_type=jnp.float32,
                  precision=lax.Precision.HIGHEST)
        + b1_ref[...])
    h = jnp.dot(h1, W2_ref[...], preferred_element_type=jnp.float32,
                  precision=lax.Precision.HIGHEST) + b2_ref[...]
    xp = feat_ref[...] + h
    xpre_ref[...] = xp
    s1 = jnp.sum(xp, axis=0, keepdims=True)
    s2 = jnp.sum(xp * xp, axis=0, keepdims=True)
    upd = jnp.concatenate([s1, s2, jnp.zeros((6, C_IN), jnp.float32)], axis=0)

    @pl.when(i == 0)
    def _():
        stats_ref[...] = upd

    @pl.when(i > 0)
    def _():
        stats_ref[...] = stats_ref[...] + upd


def _gib(coord, fcnbr, feat, dirs, Wenc, benc, W1, b1, W2, b2):
    M = coord.shape[0]
    g = M // GIB_B
    return pl.pallas_call(
        _gib_body,
        grid=(g,),
        in_specs=[
            pl.BlockSpec((GIB_B, 3), lambda i: (i, 0)),
            pl.BlockSpec((GIB_B, K, 2 * C_IN), lambda i: (i, 0, 0)),
            pl.BlockSpec((GIB_B, C_IN), lambda i: (i, 0)),
            pl.BlockSpec((N_OBS, 3), lambda i: (0, 0)),
            pl.BlockSpec((C_IN, C_ENC), lambda i: (0, 0)),
            pl.BlockSpec((1, C_ENC), lambda i: (0, 0)),
            pl.BlockSpec((C_HID, C_HID), lambda i: (0, 0)),
            pl.BlockSpec((1, C_HID), lambda i: (0, 0)),
            pl.BlockSpec((C_HID, C_IN), lambda i: (0, 0)),
            pl.BlockSpec((1, C_IN), lambda i: (0, 0)),
        ],
        out_specs=[
            pl.BlockSpec((GIB_B, C_IN), lambda i: (i, 0)),
            pl.BlockSpec((8, C_IN), lambda i: (0, 0)),
            pl.BlockSpec((GIB_B, K, 8), lambda i: (i, 0, 0)),
        ],
        out_shape=[
            jax.ShapeDtypeStruct((M, C_IN), jnp.float32),
            jax.ShapeDtypeStruct((8, C_IN), jnp.float32),
            jax.ShapeDtypeStruct((M, K, 8), jnp.float32),
        ],
    )(coord, fcnbr, feat, dirs, Wenc, benc, W1, b1, W2, b2)


def _bn(v, statsA_ref, statsB_ref, g_ref, b_ref):
    s = statsA_ref[...] + statsB_ref[...]
    mu = s[0:1, :] / N
    var = s[1:2, :] / N - mu * mu
    return (v - mu) * lax.rsqrt(var + 1e-5) * g_ref[...] + b_ref[...]


def _qkv_body(xpre_ref, statsA_ref, statsB_ref, g1_ref, be1_ref,
              Wqkv_ref, bqkv_ref, x_ref, q_ref, kv_ref):
    x = jax.nn.gelu(_bn(xpre_ref[...], statsA_ref, statsB_ref, g1_ref, be1_ref))
    x_ref[...] = x
    qkv = jnp.dot(x, Wqkv_ref[...], preferred_element_type=jnp.float32,
                  precision=lax.Precision.HIGHEST)
    qkv = qkv + bqkv_ref[...]
    q_ref[...] = qkv[:, 0:C_IN]
    kv_ref[...] = qkv[:, C_IN:3 * C_IN]


def _qkv(xpre, statsA, statsB, g1, be1, Wqkv, bqkv):
    M = xpre.shape[0]
    g = M // ROW_B
    return pl.pallas_call(
        _qkv_body,
        grid=(g,),
        in_specs=[
            pl.BlockSpec((ROW_B, C_IN), lambda i: (i, 0)),
            pl.BlockSpec((8, C_IN), lambda i: (0, 0)),
            pl.BlockSpec((8, C_IN), lambda i: (0, 0)),
            pl.BlockSpec((1, C_IN), lambda i: (0, 0)),
            pl.BlockSpec((1, C_IN), lambda i: (0, 0)),
            pl.BlockSpec((C_IN, 3 * C_IN), lambda i: (0, 0)),
            pl.BlockSpec((1, 3 * C_IN), lambda i: (0, 0)),
        ],
        out_specs=[
            pl.BlockSpec((ROW_B, C_IN), lambda i: (i, 0)),
            pl.BlockSpec((ROW_B, C_IN), lambda i: (i, 0)),
            pl.BlockSpec((ROW_B, 2 * C_IN), lambda i: (i, 0)),
        ],
        out_shape=[
            jax.ShapeDtypeStruct((M, C_IN), jnp.float32),
            jax.ShapeDtypeStruct((M, C_IN), jnp.float32),
            jax.ShapeDtypeStruct((M, 2 * C_IN), jnp.float32),
        ],
    )(xpre, statsA, statsB, g1, be1, Wqkv, bqkv)


def _attn_body(rel_ref, q_ref, kvnbr_ref,
               Wpe8_ref, Wwe_ref, bwe_ref, Wo_ref, bo_ref,
               Wfc_ref, bfc_ref, fcp_ref, stats_ref):
    i = pl.program_id(0)
    B = GIB_B
    BK = GIB_B * K
    kvn = kvnbr_ref[...]
    pe = jnp.dot(rel_ref[...], Wpe8_ref[...],
                 preferred_element_type=jnp.float32,
                 precision=lax.Precision.HIGHEST)
    qw = jnp.dot(q_ref[...], Wwe_ref[...],
                 preferred_element_type=jnp.float32,
                 precision=lax.Precision.HIGHEST) + bwe_ref[...]
    lk = jnp.dot(pe - kvn[:, 0:C_IN], Wwe_ref[...],
                 preferred_element_type=jnp.float32,
                 precision=lax.Precision.HIGHEST)
    logits = lk.reshape(B, K, GROUPS) + qw[:, None, :]
    m = jnp.max(logits, axis=1, keepdims=True)
    e = jnp.exp(logits - m)
    attn = e / jnp.sum(e, axis=1, keepdims=True)
    lane_i = lax.broadcasted_iota(jnp.int32, (GROUPS, C_IN), 1)
    grp_i = lax.broadcasted_iota(jnp.int32, (GROUPS, C_IN), 0)
    expand = (lane_i // (C_IN // GROUPS) == grp_i).astype(jnp.float32)
    attn_rep = jnp.dot(attn.reshape(BK, GROUPS), expand,
                       preferred_element_type=jnp.float32,
                       precision=lax.Precision.HIGHEST)
    prod = attn_rep * (kvn[:, C_IN:2 * C_IN] + pe)
    agg = jnp.sum(prod.reshape(B, K, C_IN), axis=1)
    agg = jnp.dot(agg, Wo_ref[...], preferred_element_type=jnp.float32,
                  precision=lax.Precision.HIGHEST)
    agg = agg + bo_ref[...]
    fcp = jnp.dot(agg, Wfc_ref[...], preferred_element_type=jnp.float32,
                  precision=lax.Precision.HIGHEST)
    fcp = fcp + bfc_ref[...]
    fcp_ref[...] = fcp
    s1 = jnp.sum(fcp, axis=0, keepdims=True)
    s2 = jnp.sum(fcp * fcp, axis=0, keepdims=True)
    upd = jnp.concatenate([s1, s2, jnp.zeros((6, C_IN), jnp.float32)], axis=0)

    @pl.when(i == 0)
    def _():
        stats_ref[...] = upd

    @pl.when(i > 0)
    def _():
        stats_ref[...] = stats_ref[...] + upd


def _attn(rel_flat, q, kvnbr_flat, Wpe8, Wwe, bwe, Wo, bo, Wfc, bfc):
    M = q.shape[0]
    g = M // GIB_B
    BK = GIB_B * K
    return pl.pallas_call(
        _attn_body,
        grid=(g,),
        in_specs=[
            pl.BlockSpec((BK, 8), lambda i: (i, 0)),
            pl.BlockSpec((GIB_B, C_IN), lambda i: (i, 0)),
            pl.BlockSpec((BK, 2 * C_IN), lambda i: (i, 0)),
            pl.BlockSpec((8, C_IN), lambda i: (0, 0)),
            pl.BlockSpec((C_IN, GROUPS), lambda i: (0, 0)),
            pl.BlockSpec((1, GROUPS), lambda i: (0, 0)),
            pl.BlockSpec((C_IN, C_IN), lambda i: (0, 0)),
            pl.BlockSpec((1, C_IN), lambda i: (0, 0)),
            pl.BlockSpec((C_IN, C_IN), lambda i: (0, 0)),
            pl.BlockSpec((1, C_IN), lambda i: (0, 0)),
        ],
        out_specs=[
            pl.BlockSpec((GIB_B, C_IN), lambda i: (i, 0)),
            pl.BlockSpec((8, C_IN), lambda i: (0, 0)),
        ],
        out_shape=[
            jax.ShapeDtypeStruct((M, C_IN), jnp.float32),
            jax.ShapeDtypeStruct((8, C_IN), jnp.float32),
        ],
    )(rel_flat, q, kvnbr_flat, Wpe8, Wwe, bwe, Wo, bo, Wfc, bfc)


def _sota_body(x_ref, fcp_ref, statsA_ref, statsB_ref, gfc_ref, befc_ref,
               Ws1_ref, bs1_ref, Ws2_ref, bs2_ref, zp_ref, stats3_ref):
    i = pl.program_id(0)
    y = x_ref[...] + jax.nn.gelu(
        _bn(fcp_ref[...], statsA_ref, statsB_ref, gfc_ref, befc_ref))
    z1 = jax.nn.gelu(
        jnp.dot(y, Ws1_ref[...], preferred_element_type=jnp.float32,
                  precision=lax.Precision.HIGHEST)
        + bs1_ref[...])
    zp = jnp.dot(z1, Ws2_ref[...], preferred_element_type=jnp.float32,
                  precision=lax.Precision.HIGHEST)
    zp = zp + bs2_ref[...]
    zp_ref[...] = zp
    s1 = jnp.sum(zp, axis=0, keepdims=True)
    s2 = jnp.sum(zp * zp, axis=0, keepdims=True)
    upd = jnp.concatenate([s1, s2, jnp.zeros((6, C_IN), jnp.float32)], axis=0)

    @pl.when(i == 0)
    def _():
        stats3_ref[...] = upd

    @pl.when(i > 0)
    def _():
        stats3_ref[...] = stats3_ref[...] + upd


def _sota(x, fcp, statsA, statsB, gfc, befc, Ws1, bs1, Ws2, bs2):
    M = x.shape[0]
    g = M // ROW_B
    return pl.pallas_call(
        _sota_body,
        grid=(g,),
        in_specs=[
            pl.BlockSpec((ROW_B, C_IN), lambda i: (i, 0)),
            pl.BlockSpec((ROW_B, C_IN), lambda i: (i, 0)),
            pl.BlockSpec((8, C_IN), lambda i: (0, 0)),
            pl.BlockSpec((8, C_IN), lambda i: (0, 0)),
            pl.BlockSpec((1, C_IN), lambda i: (0, 0)),
            pl.BlockSpec((1, C_IN), lambda i: (0, 0)),
            pl.BlockSpec((C_IN, C_IN), lambda i: (0, 0)),
            pl.BlockSpec((1, C_IN), lambda i: (0, 0)),
            pl.BlockSpec((C_IN, C_IN), lambda i: (0, 0)),
            pl.BlockSpec((1, C_IN), lambda i: (0, 0)),
        ],
        out_specs=[
            pl.BlockSpec((ROW_B, C_IN), lambda i: (i, 0)),
            pl.BlockSpec((8, C_IN), lambda i: (0, 0)),
        ],
        out_shape=[
            jax.ShapeDtypeStruct((M, C_IN), jnp.float32),
            jax.ShapeDtypeStruct((8, C_IN), jnp.float32),
        ],
    )(x, fcp, statsA, statsB, gfc, befc, Ws1, bs1, Ws2, bs2)


def _final_body(zp_ref, statsA_ref, statsB_ref, g2_ref, be2_ref, z_ref):
    z_ref[...] = jax.nn.gelu(
        _bn(zp_ref[...], statsA_ref, statsB_ref, g2_ref, be2_ref))


def _final(zp, statsA, statsB, g2, be2):
    M = zp.shape[0]
    g = M // ROW_B
    return pl.pallas_call(
        _final_body,
        grid=(g,),
        in_specs=[
            pl.BlockSpec((ROW_B, C_IN), lambda i: (i, 0)),
            pl.BlockSpec((8, C_IN), lambda i: (0, 0)),
            pl.BlockSpec((8, C_IN), lambda i: (0, 0)),
            pl.BlockSpec((1, C_IN), lambda i: (0, 0)),
            pl.BlockSpec((1, C_IN), lambda i: (0, 0)),
        ],
        out_specs=pl.BlockSpec((ROW_B, C_IN), lambda i: (i, 0)),
        out_shape=jax.ShapeDtypeStruct((M, C_IN), jnp.float32),
    )(zp, statsA, statsB, g2, be2)


def kernel(coord, feat, offset, gib_dirs, W_enc, b_enc, W1, b1, W2, b2, g1, be1,
           Wqkv, bqkv, Wpe, bpe, Wwe, bwe, Wo, bo, Wfc, bfc, gfc, befc,
           Ws1, bs1, Ws2, bs2, g2, be2):
    r1 = lambda v: v.reshape(1, -1)

    coordT = jnp.full((8, NP_PAD), 1.0e8, jnp.float32)
    coordT = coordT.at[0:3, 0:N].set(coord.T)

    fc = jnp.concatenate([feat, jnp.pad(coord, ((0, 0), (0, C_IN - 3)))], axis=1)
    Wpe8 = jnp.concatenate(
        [Wpe, bpe.reshape(1, C_IN), jnp.zeros((4, C_IN), jnp.float32)], axis=0)
    n_rows = HB_PAD // K

    def flat_idx(idx):
        return jnp.zeros((HB_PAD,), jnp.int32).at[0:NH * K].set(idx.reshape(-1))

    halves = [(coord[0:NH], feat[0:NH]), (coord[NH:N], feat[NH:N])]
    idxf, fcn = [], []
    for ch, _ in halves:
        idxf.append(flat_idx(_knn(coordT, ch)))
    for i in range(2):
        fcn.append(_sc_gather(fc, idxf[i], chunk=160).reshape(n_rows, K, 2 * C_IN))

    gib_out = []
    for i, (ch, fh) in enumerate(halves):
        gib_out.append(_gib(ch, fcn[i], fh, gib_dirs,
                            W_enc, r1(b_enc), W1, r1(b1), W2, r1(b2)))
    (xpreA, s1A, relA), (xpreB, s1B, relB) = gib_out

    xA, qA, kvA = _qkv(xpreA, s1A, s1B, r1(g1), r1(be1), Wqkv, r1(bqkv))
    xB, qB, kvB = _qkv(xpreB, s1A, s1B, r1(g1), r1(be1), Wqkv, r1(bqkv))
    kv = jnp.concatenate([kvA, kvB], axis=0)

    kvnA = _sc_gather(kv, idxf[0], chunk=160)
    kvnB = _sc_gather(kv, idxf[1], chunk=160)

    fcpA, s2A = _attn(relA.reshape(NH * K, 8), qA, kvnA, Wpe8,
                      Wwe, r1(bwe), Wo, r1(bo), Wfc, r1(bfc))
    fcpB, s2B = _attn(relB.reshape(NH * K, 8), qB, kvnB, Wpe8,
                      Wwe, r1(bwe), Wo, r1(bo), Wfc, r1(bfc))

    zpA, s3A = _sota(xA, fcpA, s2A, s2B, r1(gfc), r1(befc),
                     Ws1, r1(bs1), Ws2, r1(bs2))
    zpB, s3B = _sota(xB, fcpB, s2A, s2B, r1(gfc), r1(befc),
                     Ws1, r1(bs1), Ws2, r1(bs2))
    zA = _final(zpA, s3A, s3B, r1(g2), r1(be2))
    zB = _final(zpB, s3A, s3B, r1(g2), r1(be2))
    z = jnp.concatenate([zA, zB], axis=0)
    return (coord, z, offset)
